# Initial kernel scaffold; baseline (speedup 1.0000x reference)
import jax
import jax.numpy as jnp
from jax.experimental import pallas as pl


def kernel(x, edge_index, batch, W0, b0, W1, b1, W2, b2, W_fc, b_fc):
    raise NotImplementedError("write your pallas kernel here")


# speedup vs baseline: 14.1770x; 14.1770x over previous
import functools

import jax
import jax.numpy as jnp
from jax import lax
from jax.experimental import pallas as pl
from jax.experimental.pallas import tpu as pltpu
from jax.experimental.pallas import tpu_sc as plsc

N = 10000
E = 320000
D = 128
G = 64
NC, NS = 2, 16
NW = NC * NS
NPAD = 10240
RPT = NPAD // NS
EPW = E // NW
CH = 128
NFULL = EPW // CH
TAIL = EPW - NFULL * CH


def _sc_mesh():
    return plsc.VectorSubcoreMesh(
        core_axis_name="c", subcore_axis_name="s", num_cores=NC, num_subcores=NS
    )


@functools.partial(
    pl.kernel,
    out_type=jax.ShapeDtypeStruct((NC * NPAD,), jnp.float32),
    mesh=_sc_mesh(),
    scratch_types=[
        pltpu.VMEM((CH,), jnp.int32),
        pltpu.VMEM((TAIL,), jnp.int32),
        pltpu.VMEM((CH,), jnp.float32),
        pltpu.VMEM((RPT,), jnp.float32),
        pltpu.VMEM_SHARED((NPAD,), jnp.float32),
    ],
)
def _deg_kernel(dst_hbm, out_hbm, idx_v, idx_t, ones_v, stage_v, acc_sh):
    c = lax.axis_index("c")
    s = lax.axis_index("s")
    wid = c * NS + s

    for j in range(CH // 16):
        ones_v[pl.ds(j * 16, 16)] = jnp.ones((16,), jnp.float32)

    def zfill(j, carry):
        stage_v[pl.ds(j * 16, 16)] = jnp.zeros((16,), jnp.float32)
        return carry

    lax.fori_loop(0, RPT // 16, zfill, 0)
    pltpu.sync_copy(stage_v, acc_sh.at[pl.ds(s * RPT, RPT)])
    plsc.subcore_barrier()

    base = wid * EPW

    def body(i, carry):
        pltpu.sync_copy(dst_hbm.at[pl.ds(base + i * CH, CH)], idx_v)
        pltpu.sync_copy(ones_v, acc_sh.at[idx_v], add=True)
        return carry

    lax.fori_loop(0, NFULL, body, 0)
    pltpu.sync_copy(dst_hbm.at[pl.ds(base + NFULL * CH, TAIL)], idx_t)
    pltpu.sync_copy(ones_v.at[pl.ds(0, TAIL)], acc_sh.at[idx_t], add=True)

    plsc.subcore_barrier()
    pltpu.sync_copy(
        acc_sh.at[pl.ds(s * RPT, RPT)],
        out_hbm.at[pl.ds(c * NPAD + s * RPT, RPT)],
    )


@functools.partial(
    pl.kernel,
    out_type=jax.ShapeDtypeStruct((NC * NPAD, D), jnp.float32),
    mesh=_sc_mesh(),
    scratch_types=[
        pltpu.VMEM((CH,), jnp.int32),
        pltpu.VMEM((CH,), jnp.int32),
        pltpu.VMEM((TAIL,), jnp.int32),
        pltpu.VMEM((TAIL,), jnp.int32),
        pltpu.VMEM((CH, D), jnp.float32),
        pltpu.VMEM((TAIL, D), jnp.float32),
        pltpu.VMEM_SHARED((NPAD, D), jnp.float32),
        pltpu.SemaphoreType.DMA,
    ],
)
def _scat_kernel(hs_hbm, src_hbm, dst_hbm, zeros_hbm, out_hbm,
                 sidx, didx, sidx_t, didx_t, rows, rows_t, acc_sh, sem):
    c = lax.axis_index("c")
    s = lax.axis_index("s")
    wid = c * NS + s

    pltpu.sync_copy(zeros_hbm.at[pl.ds(s * RPT, RPT)], acc_sh.at[pl.ds(s * RPT, RPT)])
    plsc.subcore_barrier()

    base = wid * EPW

    def body(i, carry):
        off = base + i * CH
        pltpu.sync_copy(src_hbm.at[pl.ds(off, CH)], sidx)
        pltpu.sync_copy(dst_hbm.at[pl.ds(off, CH)], didx)
        pltpu.async_copy(hs_hbm.at[sidx], rows, sem).wait()
        pltpu.sync_copy(rows, acc_sh.at[didx], add=True)
        return carry

    lax.fori_loop(0, NFULL, body, 0)

    off = base + NFULL * CH
    pltpu.sync_copy(src_hbm.at[pl.ds(off, TAIL)], sidx_t)
    pltpu.sync_copy(dst_hbm.at[pl.ds(off, TAIL)], didx_t)
    pltpu.async_copy(hs_hbm.at[sidx_t], rows_t, sem).wait()
    pltpu.sync_copy(rows_t, acc_sh.at[didx_t], add=True)

    plsc.subcore_barrier()
    pltpu.sync_copy(
        acc_sh.at[pl.ds(s * RPT, RPT)],
        out_hbm.at[pl.ds(c * NPAD + s * RPT, RPT)],
    )


def _prep_body(x_ref, w_ref, d0_ref, d1_ref, dis_ref, hs_ref):
    dis = lax.rsqrt(d0_ref[...] + d1_ref[...] + 1.0)
    dis_ref[...] = dis
    hs_ref[...] = jnp.dot(x_ref[...], w_ref[...],
                          preferred_element_type=jnp.float32) * dis


_prep = pl.pallas_call(
    _prep_body,
    out_shape=(
        jax.ShapeDtypeStruct((N, 1), jnp.float32),
        jax.ShapeDtypeStruct((N, D), jnp.float32),
    ),
)


def _comb_body(scat_ref, hsp_ref, dis_ref, b_ref, w_ref, out_ref):
    agg = scat_ref[0:N, :] + scat_ref[NPAD:NPAD + N, :] + hsp_ref[...]
    xn = jnp.maximum(agg * dis_ref[...] + b_ref[...], 0.0)
    out_ref[...] = jnp.dot(xn, w_ref[...],
                           preferred_element_type=jnp.float32) * dis_ref[...]


_comb = pl.pallas_call(
    _comb_body,
    out_shape=jax.ShapeDtypeStruct((N, D), jnp.float32),
)


def _final_body(scat_ref, hsp_ref, dis_ref, b_ref, batch_ref, wfc_ref, bfc_ref,
                out_ref):
    h3 = (scat_ref[0:N, :] + scat_ref[NPAD:NPAD + N, :] + hsp_ref[...]) \
        * dis_ref[...] + b_ref[...]
    gids = lax.broadcasted_iota(jnp.int32, (G, N), 0)
    onehot = (gids == batch_ref[...]).astype(jnp.float32)
    sums = jnp.dot(onehot, h3, preferred_element_type=jnp.float32)
    counts = jnp.sum(onehot, axis=1, keepdims=True)
    pooled = sums / jnp.maximum(counts, 1.0)
    out_ref[...] = jnp.dot(pooled, wfc_ref[...],
                           preferred_element_type=jnp.float32) + bfc_ref[...]


_final = pl.pallas_call(
    _final_body,
    out_shape=jax.ShapeDtypeStruct((G, 1), jnp.float32),
)


def kernel(x, edge_index, batch, W0, b0, W1, b1, W2, b2, W_fc, b_fc):
    src = edge_index[0]
    dst = edge_index[1]
    zeros_big = jnp.zeros((NPAD, D), jnp.float32)

    degp = _deg_kernel(dst)
    d0 = degp[0:N].reshape(N, 1)
    d1 = degp[NPAD:NPAD + N].reshape(N, 1)

    dis, hs = _prep(x, W0, d0, d1)

    scat = _scat_kernel(hs, src, dst, zeros_big)
    hs = _comb(scat, hs, dis, b0.reshape(1, D), W1)

    scat = _scat_kernel(hs, src, dst, zeros_big)
    hs = _comb(scat, hs, dis, b1.reshape(1, D), W2)

    scat = _scat_kernel(hs, src, dst, zeros_big)
    out = _final(scat, hs, dis, b2.reshape(1, D), batch.reshape(1, N),
                 W_fc, b_fc.reshape(1, 1))
    return out.reshape(G)


# speedup vs baseline: 17.4833x; 1.2332x over previous
import functools

import jax
import jax.numpy as jnp
from jax import lax
from jax.experimental import pallas as pl
from jax.experimental.pallas import tpu as pltpu
from jax.experimental.pallas import tpu_sc as plsc

N = 10000
E = 320000
D = 128
G = 64
NC, NS = 2, 16
NW = NC * NS
NPAD = 10240
RPT = NPAD // NS
EPW = E // NW
CH = 128
NFULL = EPW // CH
TAIL = EPW - NFULL * CH
NBUF = 3
GAH = 2
EBN = 6
CHS = 96
NCH = 108
EPWP = NCH * CHS
NPS = 10112
RPS = NPS // NS


def _sc_mesh():
    return plsc.VectorSubcoreMesh(
        core_axis_name="c", subcore_axis_name="s", num_cores=NC, num_subcores=NS
    )


@functools.partial(
    pl.kernel,
    out_type=jax.ShapeDtypeStruct((NC * NPAD,), jnp.float32),
    mesh=_sc_mesh(),
    scratch_types=[
        pltpu.VMEM((CH,), jnp.int32),
        pltpu.VMEM((TAIL,), jnp.int32),
        pltpu.VMEM((CH,), jnp.float32),
        pltpu.VMEM((RPT,), jnp.float32),
        pltpu.VMEM_SHARED((NPAD,), jnp.float32),
    ],
)
def _deg_kernel(dst_hbm, out_hbm, idx_v, idx_t, ones_v, stage_v, acc_sh):
    c = lax.axis_index("c")
    s = lax.axis_index("s")
    wid = c * NS + s

    for j in range(CH // 16):
        ones_v[pl.ds(j * 16, 16)] = jnp.ones((16,), jnp.float32)

    def zfill(j, carry):
        stage_v[pl.ds(j * 16, 16)] = jnp.zeros((16,), jnp.float32)
        return carry

    lax.fori_loop(0, RPT // 16, zfill, 0)
    pltpu.sync_copy(stage_v, acc_sh.at[pl.ds(s * RPT, RPT)])
    plsc.subcore_barrier()

    base = wid * EPW

    def body(i, carry):
        pltpu.sync_copy(dst_hbm.at[pl.ds(base + i * CH, CH)], idx_v)
        pltpu.sync_copy(ones_v, acc_sh.at[idx_v], add=True)
        return carry

    lax.fori_loop(0, NFULL, body, 0)
    pltpu.sync_copy(dst_hbm.at[pl.ds(base + NFULL * CH, TAIL)], idx_t)
    pltpu.sync_copy(ones_v.at[pl.ds(0, TAIL)], acc_sh.at[idx_t], add=True)

    plsc.subcore_barrier()
    pltpu.sync_copy(
        acc_sh.at[pl.ds(s * RPT, RPT)],
        out_hbm.at[pl.ds(c * NPAD + s * RPT, RPT)],
    )


@functools.partial(
    pl.kernel,
    out_type=jax.ShapeDtypeStruct((NC * NPS, D), jnp.float32),
    mesh=_sc_mesh(),
    scratch_types=[
        [pltpu.VMEM((2, CHS), jnp.int32) for _ in range(EBN)],
        [pltpu.VMEM((CHS, D), jnp.float32) for _ in range(NBUF)],
        [pltpu.VMEM((CHS,), jnp.float32) for _ in range(NBUF)],
        [pltpu.VMEM((CHS,), jnp.float32) for _ in range(NBUF)],
        pltpu.VMEM_SHARED((NPS, D), jnp.float32),
        [pltpu.SemaphoreType.DMA for _ in range(EBN)],
        [pltpu.SemaphoreType.DMA for _ in range(NBUF)],
        [pltpu.SemaphoreType.DMA for _ in range(NBUF)],
        [pltpu.SemaphoreType.DMA for _ in range(NBUF)],
    ],
)
def _scat_kernel(h_hbm, dis_hbm, eidx_hbm, zeros_hbm, out_hbm,
                 ebuf, rows, disS, disD, acc_sh, isems, gsems, nsems, ssems):
    c = lax.axis_index("c")
    s = lax.axis_index("s")
    wid = c * NS + s
    ebase = wid * NCH

    pltpu.sync_copy(zeros_hbm.at[pl.ds(s * RPS, RPS)], acc_sh.at[pl.ds(s * RPS, RPS)])

    def start_gathers(ch_ref, e, b):
        pltpu.async_copy(h_hbm.at[ebuf[e].at[0]], rows[b], gsems[b])
        pltpu.async_copy(dis_hbm.at[ebuf[e].at[0]], disS[b], nsems[b])
        pltpu.async_copy(dis_hbm.at[ebuf[e].at[1]], disD[b], nsems[b])

    def wait_gathers(e, b):
        pltpu.make_async_copy(h_hbm.at[ebuf[e].at[0]], rows[b], gsems[b]).wait()
        pltpu.make_async_copy(dis_hbm.at[ebuf[e].at[0]], disS[b], nsems[b]).wait()
        pltpu.make_async_copy(dis_hbm.at[ebuf[e].at[1]], disD[b], nsems[b]).wait()

    def scale(b):
        def gbody(j, carry):
            sl16 = pl.ds(j * 16, 16)
            nv = disS[b][sl16] * disD[b][sl16]
            for i in range(16):
                nrv = lax.gather(
                    nv, jnp.full((16, 1), i, jnp.int32),
                    lax.GatherDimensionNumbers(offset_dims=(),
                                               collapsed_slice_dims=(0,),
                                               start_index_map=(0,)),
                    (1,), mode=lax.GatherScatterMode.PROMISE_IN_BOUNDS)
                r = j * 16 + i
                for k in range(D // 16):
                    sl = pl.ds(k * 16, 16)
                    rows[b][r, sl] = rows[b][r, sl] * nrv
            return carry

        lax.fori_loop(0, CHS // 16, gbody, 0)

    for ch in range(GAH + 2):
        pltpu.async_copy(eidx_hbm.at[ebase + ch], ebuf[ch], isems[ch])
    for ch in range(GAH):
        pltpu.make_async_copy(eidx_hbm.at[ebase + ch], ebuf[ch], isems[ch]).wait()
        start_gathers(None, ch, ch)
    plsc.subcore_barrier()

    def outer(g, carry):
        for bb in range(EBN):
            ch = g * EBN + bb
            b = bb % NBUF
            b2 = (bb + GAH) % NBUF
            e2 = (bb + GAH) % EBN
            wait_gathers(bb, b)
            scale(b)
            pltpu.async_copy(rows[b], acc_sh.at[ebuf[bb].at[1]], ssems[b], add=True)

            @pl.when(ch < NCH - (GAH + 2))
            def _():
                pltpu.async_copy(eidx_hbm.at[ebase + ch + GAH + 2],
                                 ebuf[(bb + GAH + 2) % EBN], isems[(bb + GAH + 2) % EBN])

            @pl.when(ch >= 1)
            def _():
                pltpu.make_async_copy(
                    rows[b2], acc_sh.at[ebuf[(bb + EBN - 1) % EBN].at[1]],
                    ssems[b2]).wait()

            @pl.when(ch < NCH - GAH)
            def _():
                pltpu.make_async_copy(
                    eidx_hbm.at[ebase + ch + GAH], ebuf[e2], isems[e2]).wait()
                start_gathers(None, e2, b2)

        return carry

    lax.fori_loop(0, NCH // EBN, outer, 0)

    pltpu.make_async_copy(
        rows[(NCH - 1) % NBUF],
        acc_sh.at[ebuf[(NCH - 1) % EBN].at[1]],
        ssems[(NCH - 1) % NBUF]).wait()

    plsc.subcore_barrier()
    pltpu.sync_copy(
        acc_sh.at[pl.ds(s * RPS, RPS)],
        out_hbm.at[pl.ds(c * NPS + s * RPS, RPS)],
    )


def _dot(a, b):
    return jnp.dot(a, b, preferred_element_type=jnp.float32)


def _prep_body(x_ref, w_ref, d0_ref, d1_ref, dis_ref, nself_ref, ht_ref):
    dis = lax.rsqrt(d0_ref[...] + d1_ref[...] + 1.0)
    dis_ref[...] = dis
    nself_ref[...] = dis * dis
    ht_ref[...] = _dot(x_ref[...], w_ref[...])


_prep = pl.pallas_call(
    _prep_body,
    out_shape=(
        jax.ShapeDtypeStruct((N, 1), jnp.float32),
        jax.ShapeDtypeStruct((N, 1), jnp.float32),
        jax.ShapeDtypeStruct((N, D), jnp.float32),
    ),
)


def _comb_body(scat_ref, ht_ref, nself_ref, b_ref, w_ref, out_ref):
    agg = scat_ref[0:N, :] + scat_ref[NPS:NPS + N, :] + ht_ref[...] * nself_ref[...]
    xn = jnp.maximum(agg + b_ref[...], 0.0)
    out_ref[...] = _dot(xn, w_ref[...])


_comb = pl.pallas_call(
    _comb_body,
    out_shape=jax.ShapeDtypeStruct((N, D), jnp.float32),
)


def _final_body(scat_ref, ht_ref, nself_ref, b_ref, batch_ref, wfc_ref, bfc_ref,
                out_ref):
    h3 = scat_ref[0:N, :] + scat_ref[NPS:NPS + N, :] \
        + ht_ref[...] * nself_ref[...] + b_ref[...]
    gids = lax.broadcasted_iota(jnp.int32, (G, N), 0)
    onehot = (gids == batch_ref[...]).astype(jnp.float32)
    sums = _dot(onehot, h3)
    counts = jnp.sum(onehot, axis=1, keepdims=True)
    pooled = sums / jnp.maximum(counts, 1.0)
    out_ref[...] = _dot(pooled, wfc_ref[...]) + bfc_ref[...]


_final = pl.pallas_call(
    _final_body,
    out_shape=jax.ShapeDtypeStruct((G, 1), jnp.float32),
)


def kernel(x, edge_index, batch, W0, b0, W1, b1, W2, b2, W_fc, b_fc):
    src = edge_index[0]
    dst = edge_index[1]
    zeros_big = jnp.zeros((NPS, D), jnp.float32)

    npad_e = EPWP - EPW
    pad_s = (jnp.arange(npad_e, dtype=jnp.int32) * 97 % N)[None, :].repeat(NW, 0)
    pad_d = (N + jnp.arange(npad_e, dtype=jnp.int32) % (NPS - N))[None, :].repeat(NW, 0)
    src_p = jnp.concatenate([src.reshape(NW, EPW), pad_s], axis=1).reshape(NW, NCH, CHS)
    dst_p = jnp.concatenate([dst.reshape(NW, EPW), pad_d], axis=1).reshape(NW, NCH, CHS)
    eidx = jnp.stack([src_p, dst_p], axis=2).reshape(NW * NCH, 2, CHS)

    degp = _deg_kernel(dst)
    d0 = degp[0:N].reshape(N, 1)
    d1 = degp[NPAD:NPAD + N].reshape(N, 1)

    dis, nself, ht = _prep(x, W0, d0, d1)
    dis_flat = jnp.concatenate([dis[:, 0], jnp.zeros((NPS - N,), jnp.float32)])

    scat = _scat_kernel(ht, dis_flat, eidx, zeros_big)
    ht = _comb(scat, ht, nself, b0.reshape(1, D), W1)

    scat = _scat_kernel(ht, dis_flat, eidx, zeros_big)
    ht = _comb(scat, ht, nself, b1.reshape(1, D), W2)

    scat = _scat_kernel(ht, dis_flat, eidx, zeros_big)
    out = _final(scat, ht, nself, b2.reshape(1, D), batch.reshape(1, N),
                 W_fc, b_fc.reshape(1, 1))
    return out.reshape(G)
